# baseline (speedup 1.0000x reference)
import functools

import jax
import jax.numpy as jnp
from jax import lax
from jax.experimental import pallas as pl
from jax.experimental.pallas import tpu as pltpu
from jax.experimental.pallas import tpu_sc as plsc

L = 16


def kernel(user_indices, item_indices, user_factors, item_factors,
           user_bias, item_bias, global_bias):
    B = user_indices.shape[0]
    D = user_factors.shape[1]

    mesh = plsc.VectorSubcoreMesh(core_axis_name="c", subcore_axis_name="s")
    nc, ns = mesh.num_cores, mesh.num_subcores
    nw = nc * ns
    b_per_w = B // nw

    @functools.partial(
        pl.kernel,
        out_type=jax.ShapeDtypeStruct((B,), jnp.float32),
        mesh=mesh,
        compiler_params=pltpu.CompilerParams(
            needs_layout_passes=False, use_tc_tiling_on_sc=False),
        scratch_types=[
            pltpu.VMEM((b_per_w,), jnp.int32),
            pltpu.VMEM((b_per_w,), jnp.int32),
            pltpu.VMEM((b_per_w,), jnp.int32),
            pltpu.VMEM((b_per_w,), jnp.int32),
            pltpu.VMEM((b_per_w, 32), jnp.float32),
            pltpu.VMEM((b_per_w, 32), jnp.float32),
            pltpu.VMEM((b_per_w, L), jnp.float32),
            pltpu.VMEM((b_per_w, L), jnp.float32),
            pltpu.VMEM((b_per_w,), jnp.float32),
            pltpu.VMEM((L,), jnp.float32),
            pltpu.SemaphoreType.DMA,
        ],
    )
    def mf(uidx_hbm, iidx_hbm, uf_hbm, if_hbm, ub_hbm, ib_hbm, gb_hbm,
           out_hbm, uidx_v, iidx_v, uidx4_v, iidx4_v, urows_v, irows_v,
           ubg_v, ibg_v, out_v, gb_v, sem):
        wid = lax.axis_index("s") * nc + lax.axis_index("c")
        base = wid * b_per_w
        pltpu.sync_copy(uidx_hbm.at[pl.ds(base, b_per_w)], uidx_v)
        pltpu.sync_copy(iidx_hbm.at[pl.ds(base, b_per_w)], iidx_v)
        pltpu.sync_copy(gb_hbm, gb_v)
        c1 = pltpu.async_copy(uf_hbm.at[uidx_v], urows_v, sem)
        c2 = pltpu.async_copy(if_hbm.at[iidx_v], irows_v, sem)

        def shift(g, carry):
            b0 = g * L
            uidx4_v[pl.ds(b0, L)] = lax.shift_right_logical(
                uidx_v[pl.ds(b0, L)], 4)
            iidx4_v[pl.ds(b0, L)] = lax.shift_right_logical(
                iidx_v[pl.ds(b0, L)], 4)
            return carry

        lax.fori_loop(0, b_per_w // L, shift, 0)
        c3 = pltpu.async_copy(ub_hbm.at[uidx4_v], ubg_v, sem)
        c4 = pltpu.async_copy(ib_hbm.at[iidx4_v], ibg_v, sem)
        c1.wait()
        c2.wait()
        c3.wait()
        c4.wait()

        lanes = lax.iota(jnp.int32, L)
        gb = gb_v[pl.ds(0, L)]
        mask15 = jnp.full((L,), 15, jnp.int32)

        def group(g, carry):
            b0 = g * L
            row = b0 + lanes
            ucol = uidx_v[pl.ds(b0, L)] & mask15
            icol = iidx_v[pl.ds(b0, L)] & mask15
            acc = (plsc.load_gather(ubg_v, [row, ucol])
                   + plsc.load_gather(ibg_v, [row, icol]))
            for d in range(D):
                col = jnp.full((L,), d, jnp.int32)
                acc = acc + (plsc.load_gather(urows_v, [row, col])
                             * plsc.load_gather(irows_v, [row, col]))
            out_v[pl.ds(b0, L)] = acc + gb
            return carry

        lax.fori_loop(0, b_per_w // L, group, 0)
        pltpu.sync_copy(out_v, out_hbm.at[pl.ds(base, b_per_w)])

    return mf(user_indices, item_indices, user_factors, item_factors,
              user_bias.reshape(-1, L), item_bias.reshape(-1, L),
              jnp.broadcast_to(global_bias, (L,)))
